# Initial kernel scaffold; baseline (speedup 1.0000x reference)
import jax
import jax.numpy as jnp
from jax.experimental import pallas as pl


def kernel(x, edge_index, W1, b1, W2, b2):
    raise NotImplementedError("write your pallas kernel here")


# speedup vs baseline: 18.1298x; 18.1298x over previous
import functools

import jax
import jax.numpy as jnp
from jax import lax
from jax.experimental import pallas as pl
from jax.experimental.pallas import tpu as pltpu
from jax.experimental.pallas import tpu_sc as plsc

N = 10000
E = 320000
DIN = 128
DH = 256
DOUT = 128

NC = 2
NS = 16
CHUNK = 128

CHUNKS_PER_TILE = 160
E_PAD = NS * CHUNKS_PER_TILE * CHUNK
N_CHUNKS = E_PAD // CHUNK
DEG_CHUNKS = N_CHUNKS // (NC * NS)

GROUP = 16
GROUPS = CHUNKS_PER_TILE // GROUP

ACC_ROWS = 10240
ROWS_PER_TILE = ACC_ROWS // NS
PAD_SPREAD = ACC_ROWS - N

_mesh = functools.partial(
    plsc.VectorSubcoreMesh,
    core_axis_name="c", subcore_axis_name="s", num_cores=NC, num_subcores=NS,
)


@functools.partial(
    pl.kernel,
    out_type=jax.ShapeDtypeStruct((NC * ACC_ROWS,), jnp.float32),
    mesh=_mesh(),
    scratch_types=[
        pltpu.VMEM_SHARED((ACC_ROWS,), jnp.float32),
        pltpu.VMEM((DEG_CHUNKS, CHUNK), jnp.int32),
        pltpu.VMEM((CHUNK,), jnp.float32),
    ],
)
def _deg_kernel(dst_hbm, ones_hbm, zeros_hbm, deg_out, deg_sh, idx_v, ones_v):
    c = lax.axis_index("c")
    s = lax.axis_index("s")
    w = c * NS + s
    pltpu.sync_copy(zeros_hbm, deg_sh.at[pl.ds(s * ROWS_PER_TILE, ROWS_PER_TILE)])
    pltpu.sync_copy(ones_hbm, ones_v)
    pltpu.sync_copy(dst_hbm.at[pl.ds(w * DEG_CHUNKS, DEG_CHUNKS)], idx_v)
    plsc.subcore_barrier()

    @pl.loop(0, DEG_CHUNKS)
    def _(j):
        pltpu.sync_copy(ones_v, deg_sh.at[idx_v.at[j]], add=True)

    plsc.subcore_barrier()
    sl = pl.ds(s * ROWS_PER_TILE, ROWS_PER_TILE)
    pltpu.sync_copy(deg_sh.at[sl],
                    deg_out.at[pl.ds(c * ACC_ROWS + s * ROWS_PER_TILE, ROWS_PER_TILE)])


def _make_scatter_kernel(F):
    @functools.partial(
        pl.kernel,
        out_type=jax.ShapeDtypeStruct((NC, ACC_ROWS, F), jnp.float32),
        mesh=_mesh(),
        scratch_types=[
            pltpu.VMEM_SHARED((ACC_ROWS, F), jnp.float32),
            pltpu.VMEM((GROUP, CHUNK), jnp.int32),
            pltpu.VMEM((GROUP, CHUNK), jnp.int32),
            pltpu.VMEM((CHUNK, F), jnp.float32),
        ],
    )
    def scatter_kernel(ga_hbm, gb_hbm, src_hbm, dst_hbm, zeros_hbm, out_hbm,
                       acc_sh, src_v, dst_v, rows_v):
        c = lax.axis_index("c")
        s = lax.axis_index("s")
        pltpu.sync_copy(zeros_hbm, acc_sh.at[pl.ds(s * ROWS_PER_TILE, ROWS_PER_TILE)])
        plsc.subcore_barrier()

        @pl.loop(0, GROUPS)
        def _(g):
            cb = s * CHUNKS_PER_TILE + g * GROUP
            pltpu.sync_copy(src_hbm.at[pl.ds(cb, GROUP)], src_v)
            pltpu.sync_copy(dst_hbm.at[pl.ds(cb, GROUP)], dst_v)

            @pl.loop(0, GROUP)
            def _(j):
                @pl.when(c == 0)
                def _():
                    pltpu.sync_copy(ga_hbm.at[src_v.at[j]], rows_v)

                @pl.when(c == 1)
                def _():
                    pltpu.sync_copy(gb_hbm.at[src_v.at[j]], rows_v)

                pltpu.sync_copy(rows_v, acc_sh.at[dst_v.at[j]], add=True)

        plsc.subcore_barrier()
        sl = pl.ds(s * ROWS_PER_TILE, ROWS_PER_TILE)
        pltpu.sync_copy(acc_sh.at[sl], out_hbm.at[c, sl])

    return scatter_kernel


_scatter128 = _make_scatter_kernel(DH // 2)

_CPT2 = N_CHUNKS // (NC * NS)
_GROUPS2 = _CPT2 // GROUP


@functools.partial(
    pl.kernel,
    out_type=jax.ShapeDtypeStruct((NC, ACC_ROWS, DOUT), jnp.float32),
    mesh=_mesh(),
    scratch_types=[
        pltpu.VMEM_SHARED((ACC_ROWS, DOUT), jnp.float32),
        pltpu.VMEM((GROUP, CHUNK), jnp.int32),
        pltpu.VMEM((GROUP, CHUNK), jnp.int32),
        pltpu.VMEM((CHUNK, DOUT), jnp.float32),
    ],
)
def _scatter_l2(g_hbm, src_hbm, dst_hbm, zeros_hbm, out_hbm,
                acc_sh, src_v, dst_v, rows_v):
    c = lax.axis_index("c")
    s = lax.axis_index("s")
    pltpu.sync_copy(zeros_hbm, acc_sh.at[pl.ds(s * ROWS_PER_TILE, ROWS_PER_TILE)])
    plsc.subcore_barrier()

    @pl.loop(0, _GROUPS2)
    def _(g):
        cb = (c * NS + s) * _CPT2 + g * GROUP
        pltpu.sync_copy(src_hbm.at[pl.ds(cb, GROUP)], src_v)
        pltpu.sync_copy(dst_hbm.at[pl.ds(cb, GROUP)], dst_v)

        @pl.loop(0, GROUP)
        def _(j):
            pltpu.sync_copy(g_hbm.at[src_v.at[j]], rows_v)
            pltpu.sync_copy(rows_v, acc_sh.at[dst_v.at[j]], add=True)

    plsc.subcore_barrier()
    sl = pl.ds(s * ROWS_PER_TILE, ROWS_PER_TILE)
    pltpu.sync_copy(acc_sh.at[sl], out_hbm.at[c, sl])


_ROWS_BLK = 2560
_GRID = (N + _ROWS_BLK - 1) // _ROWS_BLK


def _tc_a_body(x_ref, w_ref, deg_ref, out_ref):
    dinv = lax.rsqrt(deg_ref[0] + deg_ref[1] + 1.0)
    h = jnp.dot(x_ref[...], w_ref[...], preferred_element_type=jnp.float32)
    g = h * dinv[:, None]
    out_ref[0] = g[:, : DH // 2]
    out_ref[1] = g[:, DH // 2:]


def _tc_a(x, w1, degp):
    return pl.pallas_call(
        _tc_a_body,
        grid=(_GRID,),
        in_specs=[
            pl.BlockSpec((_ROWS_BLK, DIN), lambda i: (i, 0)),
            pl.BlockSpec((DIN, DH), lambda i: (0, 0)),
            pl.BlockSpec((NC, _ROWS_BLK), lambda i: (0, i)),
        ],
        out_specs=pl.BlockSpec((NC, _ROWS_BLK, DH // 2), lambda i: (0, i, 0)),
        out_shape=jax.ShapeDtypeStruct((NC, N, DH // 2), jnp.float32),
    )(x, w1, degp)


def _tc_b_body(acc_ref, g_ref, deg_ref, b1_ref, w2_ref, out_ref):
    dinv = lax.rsqrt(deg_ref[0] + deg_ref[1] + 1.0)
    f = jnp.concatenate([acc_ref[0] + g_ref[0], acc_ref[1] + g_ref[1]], axis=1)
    z = jnp.maximum(f * dinv[:, None] + b1_ref[...], 0.0)
    h2 = jnp.dot(z, w2_ref[...], preferred_element_type=jnp.float32)
    out_ref[...] = h2 * dinv[:, None]


def _tc_b(acc1p, g1p, degp, b1, w2):
    return pl.pallas_call(
        _tc_b_body,
        grid=(_GRID,),
        in_specs=[
            pl.BlockSpec((NC, _ROWS_BLK, DH // 2), lambda i: (0, i, 0)),
            pl.BlockSpec((NC, _ROWS_BLK, DH // 2), lambda i: (0, i, 0)),
            pl.BlockSpec((NC, _ROWS_BLK), lambda i: (0, i)),
            pl.BlockSpec((DH,), lambda i: (0,)),
            pl.BlockSpec((DH, DOUT), lambda i: (0, 0)),
        ],
        out_specs=pl.BlockSpec((_ROWS_BLK, DOUT), lambda i: (i, 0)),
        out_shape=jax.ShapeDtypeStruct((N, DOUT), jnp.float32),
    )(acc1p, g1p, degp, b1, w2)


def _tc_c_body(acc_ref, g_ref, deg_ref, b2_ref, out_ref):
    dinv = lax.rsqrt(deg_ref[0] + deg_ref[1] + 1.0)
    f = acc_ref[0] + acc_ref[1] + g_ref[...]
    out_ref[...] = f * dinv[:, None] + b2_ref[...]


def _tc_c(acc2p, g2, degp, b2):
    return pl.pallas_call(
        _tc_c_body,
        grid=(_GRID,),
        in_specs=[
            pl.BlockSpec((NC, _ROWS_BLK, DOUT), lambda i: (0, i, 0)),
            pl.BlockSpec((_ROWS_BLK, DOUT), lambda i: (i, 0)),
            pl.BlockSpec((NC, _ROWS_BLK), lambda i: (0, i)),
            pl.BlockSpec((DOUT,), lambda i: (0,)),
        ],
        out_specs=pl.BlockSpec((_ROWS_BLK, DOUT), lambda i: (i, 0)),
        out_shape=jax.ShapeDtypeStruct((N, DOUT), jnp.float32),
    )(acc2p, g2, degp, b2)


@jax.jit
def kernel(x, edge_index, W1, b1, W2, b2):
    ei = edge_index.astype(jnp.int32)
    src = ei[0]
    dst = ei[1]
    npad = E_PAD - E
    pad_i = jnp.arange(npad, dtype=jnp.int32)
    pad_src = (pad_i * 7919) % N
    pad_dst = N + pad_i % PAD_SPREAD
    srcp = jnp.concatenate([src, pad_src]).reshape(N_CHUNKS, CHUNK)
    dstp = jnp.concatenate([dst, pad_dst]).reshape(N_CHUNKS, CHUNK)

    ones128 = jnp.ones((CHUNK,), jnp.float32)
    z640 = jnp.zeros((ROWS_PER_TILE,), jnp.float32)
    z640a = jnp.zeros((ROWS_PER_TILE, DH // 2), jnp.float32)
    z640b = jnp.zeros((ROWS_PER_TILE, DOUT), jnp.float32)

    degp = _deg_kernel(dstp, ones128, z640).reshape(NC, ACC_ROWS)
    g1p = _tc_a(x, W1, degp)
    acc1p = _scatter128(g1p[0], g1p[1], srcp, dstp, z640a)
    g2 = _tc_b(acc1p, g1p, degp, b1, W2)
    acc2p = _scatter_l2(g2, srcp, dstp, z640b)
    return _tc_c(acc2p, g2, degp, b2)


# speedup vs baseline: 23.3083x; 1.2856x over previous
import functools

import jax
import jax.numpy as jnp
from jax import lax
from jax.experimental import pallas as pl
from jax.experimental.pallas import tpu as pltpu
from jax.experimental.pallas import tpu_sc as plsc

N = 10000
E = 320000
DIN = 128
DH = 256
DOUT = 128

NC = 2
NS = 16
CHUNK = 128

CHUNKS_PER_TILE = 160
E_PAD = NS * CHUNKS_PER_TILE * CHUNK
N_CHUNKS = E_PAD // CHUNK
DEG_CHUNKS = N_CHUNKS // (NC * NS)

GROUP = 16
GROUPS = CHUNKS_PER_TILE // GROUP

ACC_ROWS = 10240
ROWS_PER_TILE = ACC_ROWS // NS
PAD_SPREAD = ACC_ROWS - N

_mesh = functools.partial(
    plsc.VectorSubcoreMesh,
    core_axis_name="c", subcore_axis_name="s", num_cores=NC, num_subcores=NS,
)


@functools.partial(
    pl.kernel,
    out_type=jax.ShapeDtypeStruct((NC * ACC_ROWS,), jnp.float32),
    mesh=_mesh(),
    scratch_types=[
        pltpu.VMEM_SHARED((ACC_ROWS,), jnp.float32),
        pltpu.VMEM((DEG_CHUNKS, CHUNK), jnp.int32),
        pltpu.VMEM((CHUNK,), jnp.float32),
    ],
)
def _deg_kernel(dst_hbm, ones_hbm, zeros_hbm, deg_out, deg_sh, idx_v, ones_v):
    c = lax.axis_index("c")
    s = lax.axis_index("s")
    w = c * NS + s
    pltpu.sync_copy(zeros_hbm, deg_sh.at[pl.ds(s * ROWS_PER_TILE, ROWS_PER_TILE)])
    pltpu.sync_copy(ones_hbm, ones_v)
    pltpu.sync_copy(dst_hbm.at[pl.ds(w * DEG_CHUNKS, DEG_CHUNKS)], idx_v)
    plsc.subcore_barrier()

    @pl.loop(0, DEG_CHUNKS)
    def _(j):
        pltpu.sync_copy(ones_v, deg_sh.at[idx_v.at[j]], add=True)

    plsc.subcore_barrier()
    sl = pl.ds(s * ROWS_PER_TILE, ROWS_PER_TILE)
    pltpu.sync_copy(deg_sh.at[sl],
                    deg_out.at[pl.ds(c * ACC_ROWS + s * ROWS_PER_TILE, ROWS_PER_TILE)])


def _make_scatter_kernel(F, per_tile_chunks, core_split):
    groups = per_tile_chunks // GROUP

    @functools.partial(
        pl.kernel,
        out_type=jax.ShapeDtypeStruct((NC, ACC_ROWS, F), jnp.float32),
        mesh=_mesh(),
        scratch_types=[
            pltpu.VMEM_SHARED((ACC_ROWS, F), jnp.float32),
            pltpu.VMEM((GROUP, CHUNK), jnp.int32),
            pltpu.VMEM((GROUP, CHUNK), jnp.int32),
            pltpu.VMEM((2, CHUNK, F), jnp.float32),
            pltpu.SemaphoreType.DMA((2,)),
            pltpu.SemaphoreType.DMA((2,)),
        ],
    )
    def scatter_kernel(tab_hbm, sa_hbm, sb_hbm, dst_hbm, zeros_hbm, out_hbm,
                       acc_sh, src_v, dst_v, rows_v, gsem, ssem):
        c = lax.axis_index("c")
        s = lax.axis_index("s")
        pltpu.sync_copy(zeros_hbm, acc_sh.at[pl.ds(s * ROWS_PER_TILE, ROWS_PER_TILE)])
        if core_split:
            tile0 = (c * NS + s) * per_tile_chunks
        else:
            tile0 = s * per_tile_chunks
        plsc.subcore_barrier()

        @pl.loop(0, groups)
        def _(g):
            cb = tile0 + g * GROUP

            @pl.when(c == 0)
            def _():
                pltpu.sync_copy(sa_hbm.at[pl.ds(cb, GROUP)], src_v)

            @pl.when(c == 1)
            def _():
                pltpu.sync_copy(sb_hbm.at[pl.ds(cb, GROUP)], src_v)

            pltpu.sync_copy(dst_hbm.at[pl.ds(cb, GROUP)], dst_v)

            gd = [pltpu.async_copy(tab_hbm.at[src_v.at[0]], rows_v.at[0],
                                   gsem.at[0]), None]
            sd = [None, None]
            for jj in range(GROUP):
                b = jj & 1
                nb = 1 - b
                gd[b].wait()
                if jj + 1 < GROUP:
                    if jj >= 1:
                        sd[nb].wait()
                    gd[nb] = pltpu.async_copy(tab_hbm.at[src_v.at[jj + 1]],
                                              rows_v.at[nb], gsem.at[nb])
                sd[b] = pltpu.async_copy(rows_v.at[b], acc_sh.at[dst_v.at[jj]],
                                         ssem.at[b], add=True)
            sd[0].wait()
            sd[1].wait()

        plsc.subcore_barrier()
        sl = pl.ds(s * ROWS_PER_TILE, ROWS_PER_TILE)
        pltpu.sync_copy(acc_sh.at[sl], out_hbm.at[c, sl])

    return scatter_kernel


_scatter_l1 = _make_scatter_kernel(DH // 2, CHUNKS_PER_TILE, core_split=False)
_scatter_l2 = _make_scatter_kernel(DOUT, N_CHUNKS // (NC * NS), core_split=True)


_ROWS_BLK = 2560
_GRID = (N + _ROWS_BLK - 1) // _ROWS_BLK


def _tc_a_body(x_ref, w_ref, deg_ref, out_ref):
    dinv = lax.rsqrt(deg_ref[0] + deg_ref[1] + 1.0)
    h = jnp.dot(x_ref[...], w_ref[...], preferred_element_type=jnp.float32)
    g = h * dinv[:, None]
    out_ref[0] = g[:, : DH // 2]
    out_ref[1] = g[:, DH // 2:]


def _tc_a(x, w1, degp):
    return pl.pallas_call(
        _tc_a_body,
        grid=(_GRID,),
        in_specs=[
            pl.BlockSpec((_ROWS_BLK, DIN), lambda i: (i, 0)),
            pl.BlockSpec((DIN, DH), lambda i: (0, 0)),
            pl.BlockSpec((NC, _ROWS_BLK), lambda i: (0, i)),
        ],
        out_specs=pl.BlockSpec((NC, _ROWS_BLK, DH // 2), lambda i: (0, i, 0)),
        out_shape=jax.ShapeDtypeStruct((NC, N, DH // 2), jnp.float32),
    )(x, w1, degp)


def _tc_b_body(acc_ref, g_ref, deg_ref, b1_ref, w2_ref, out_ref):
    dinv = lax.rsqrt(deg_ref[0] + deg_ref[1] + 1.0)
    f = jnp.concatenate([acc_ref[0] + g_ref[0], acc_ref[1] + g_ref[1]], axis=1)
    z = jnp.maximum(f * dinv[:, None] + b1_ref[...], 0.0)
    h2 = jnp.dot(z, w2_ref[...], preferred_element_type=jnp.float32)
    out_ref[...] = h2 * dinv[:, None]


def _tc_b(acc1p, g1p, degp, b1, w2):
    return pl.pallas_call(
        _tc_b_body,
        grid=(_GRID,),
        in_specs=[
            pl.BlockSpec((NC, _ROWS_BLK, DH // 2), lambda i: (0, i, 0)),
            pl.BlockSpec((NC, _ROWS_BLK, DH // 2), lambda i: (0, i, 0)),
            pl.BlockSpec((NC, _ROWS_BLK), lambda i: (0, i)),
            pl.BlockSpec((DH,), lambda i: (0,)),
            pl.BlockSpec((DH, DOUT), lambda i: (0, 0)),
        ],
        out_specs=pl.BlockSpec((_ROWS_BLK, DOUT), lambda i: (i, 0)),
        out_shape=jax.ShapeDtypeStruct((N, DOUT), jnp.float32),
    )(acc1p, g1p, degp, b1, w2)


def _tc_c_body(acc_ref, g_ref, deg_ref, b2_ref, out_ref):
    dinv = lax.rsqrt(deg_ref[0] + deg_ref[1] + 1.0)
    f = acc_ref[0] + acc_ref[1] + g_ref[...]
    out_ref[...] = f * dinv[:, None] + b2_ref[...]


def _tc_c(acc2p, g2, degp, b2):
    return pl.pallas_call(
        _tc_c_body,
        grid=(_GRID,),
        in_specs=[
            pl.BlockSpec((NC, _ROWS_BLK, DOUT), lambda i: (0, i, 0)),
            pl.BlockSpec((_ROWS_BLK, DOUT), lambda i: (i, 0)),
            pl.BlockSpec((NC, _ROWS_BLK), lambda i: (0, i)),
            pl.BlockSpec((DOUT,), lambda i: (0,)),
        ],
        out_specs=pl.BlockSpec((_ROWS_BLK, DOUT), lambda i: (i, 0)),
        out_shape=jax.ShapeDtypeStruct((N, DOUT), jnp.float32),
    )(acc2p, g2, degp, b2)


@jax.jit
def kernel(x, edge_index, W1, b1, W2, b2):
    ei = edge_index.astype(jnp.int32)
    src = ei[0]
    dst = ei[1]
    npad = E_PAD - E
    pad_i = jnp.arange(npad, dtype=jnp.int32)
    pad_src = (pad_i * 7919) % N
    pad_dst = N + pad_i % PAD_SPREAD
    srcp = jnp.concatenate([src, pad_src]).reshape(N_CHUNKS, CHUNK)
    dstp = jnp.concatenate([dst, pad_dst]).reshape(N_CHUNKS, CHUNK)

    ones128 = jnp.ones((CHUNK,), jnp.float32)
    z640 = jnp.zeros((ROWS_PER_TILE,), jnp.float32)
    z640a = jnp.zeros((ROWS_PER_TILE, DH // 2), jnp.float32)
    z640b = jnp.zeros((ROWS_PER_TILE, DOUT), jnp.float32)

    srcp_hi = srcp + N

    degp = _deg_kernel(dstp, ones128, z640).reshape(NC, ACC_ROWS)
    g1p = _tc_a(x, W1, degp)
    acc1p = _scatter_l1(g1p.reshape(NC * N, DH // 2), srcp, srcp_hi, dstp, z640a)
    g2 = _tc_b(acc1p, g1p, degp, b1, W2)
    acc2p = _scatter_l2(g2, srcp, srcp, dstp, z640b)
    return _tc_c(acc2p, g2, degp, b2)


# speedup vs baseline: 24.0089x; 1.0301x over previous
import functools

import jax
import jax.numpy as jnp
from jax import lax
from jax.experimental import pallas as pl
from jax.experimental.pallas import tpu as pltpu
from jax.experimental.pallas import tpu_sc as plsc

N = 10000
E = 320000
DIN = 128
DH = 256
DOUT = 128

NC = 2
NS = 16
CHUNK = 128

CHUNKS_PER_TILE = 160
E_PAD = NS * CHUNKS_PER_TILE * CHUNK
N_CHUNKS = E_PAD // CHUNK
DEG_CHUNKS = N_CHUNKS // (NC * NS)

GROUP = 40

ACC_ROWS = 10240
ROWS_PER_TILE = ACC_ROWS // NS
PAD_SPREAD = ACC_ROWS - N

_mesh = functools.partial(
    plsc.VectorSubcoreMesh,
    core_axis_name="c", subcore_axis_name="s", num_cores=NC, num_subcores=NS,
)


@functools.partial(
    pl.kernel,
    out_type=jax.ShapeDtypeStruct((NC * ACC_ROWS,), jnp.float32),
    mesh=_mesh(),
    scratch_types=[
        pltpu.VMEM_SHARED((ACC_ROWS,), jnp.float32),
        pltpu.VMEM((DEG_CHUNKS, CHUNK), jnp.int32),
        pltpu.VMEM((CHUNK,), jnp.float32),
    ],
)
def _deg_kernel(dst_hbm, ones_hbm, zeros_hbm, deg_out, deg_sh, idx_v, ones_v):
    c = lax.axis_index("c")
    s = lax.axis_index("s")
    w = c * NS + s
    pltpu.sync_copy(zeros_hbm, deg_sh.at[pl.ds(s * ROWS_PER_TILE, ROWS_PER_TILE)])
    pltpu.sync_copy(ones_hbm, ones_v)
    pltpu.sync_copy(dst_hbm.at[pl.ds(w * DEG_CHUNKS, DEG_CHUNKS)], idx_v)
    plsc.subcore_barrier()

    @pl.loop(0, DEG_CHUNKS)
    def _(j):
        pltpu.sync_copy(ones_v, deg_sh.at[idx_v.at[j]], add=True)

    plsc.subcore_barrier()
    sl = pl.ds(s * ROWS_PER_TILE, ROWS_PER_TILE)
    pltpu.sync_copy(deg_sh.at[sl],
                    deg_out.at[pl.ds(c * ACC_ROWS + s * ROWS_PER_TILE, ROWS_PER_TILE)])


def _make_scatter_kernel(F, per_tile_chunks, core_split):
    groups = per_tile_chunks // GROUP

    @functools.partial(
        pl.kernel,
        out_type=jax.ShapeDtypeStruct((NC, ACC_ROWS, F), jnp.float32),
        mesh=_mesh(),
        scratch_types=[
            pltpu.VMEM_SHARED((ACC_ROWS, F), jnp.float32),
            pltpu.VMEM((GROUP, CHUNK), jnp.int32),
            pltpu.VMEM((GROUP, CHUNK), jnp.int32),
            pltpu.VMEM((2, CHUNK, F), jnp.float32),
            pltpu.SemaphoreType.DMA((2,)),
            pltpu.SemaphoreType.DMA((2,)),
        ],
    )
    def scatter_kernel(tab_hbm, sa_hbm, sb_hbm, dst_hbm, zeros_hbm, out_hbm,
                       acc_sh, src_v, dst_v, rows_v, gsem, ssem):
        c = lax.axis_index("c")
        s = lax.axis_index("s")
        pltpu.sync_copy(zeros_hbm, acc_sh.at[pl.ds(s * ROWS_PER_TILE, ROWS_PER_TILE)])
        if core_split:
            tile0 = (c * NS + s) * per_tile_chunks
        else:
            tile0 = s * per_tile_chunks
        plsc.subcore_barrier()

        @pl.loop(0, groups)
        def _(g):
            cb = tile0 + g * GROUP

            @pl.when(c == 0)
            def _():
                pltpu.sync_copy(sa_hbm.at[pl.ds(cb, GROUP)], src_v)

            @pl.when(c == 1)
            def _():
                pltpu.sync_copy(sb_hbm.at[pl.ds(cb, GROUP)], src_v)

            pltpu.sync_copy(dst_hbm.at[pl.ds(cb, GROUP)], dst_v)

            gd = [pltpu.async_copy(tab_hbm.at[src_v.at[0]], rows_v.at[0],
                                   gsem.at[0]), None]
            sd = [None, None]
            for jj in range(GROUP):
                b = jj & 1
                nb = 1 - b
                gd[b].wait()
                if jj + 1 < GROUP:
                    if jj >= 1:
                        sd[nb].wait()
                    gd[nb] = pltpu.async_copy(tab_hbm.at[src_v.at[jj + 1]],
                                              rows_v.at[nb], gsem.at[nb])
                sd[b] = pltpu.async_copy(rows_v.at[b], acc_sh.at[dst_v.at[jj]],
                                         ssem.at[b], add=True)
            sd[0].wait()
            sd[1].wait()

        plsc.subcore_barrier()
        sl = pl.ds(s * ROWS_PER_TILE, ROWS_PER_TILE)
        pltpu.sync_copy(acc_sh.at[sl], out_hbm.at[c, sl])

    return scatter_kernel


_scatter_l1 = _make_scatter_kernel(DH // 2, CHUNKS_PER_TILE, core_split=False)
_scatter_l2 = _make_scatter_kernel(DOUT, N_CHUNKS // (NC * NS), core_split=True)


_ROWS_BLK = 2560
_GRID = (N + _ROWS_BLK - 1) // _ROWS_BLK


def _tc_a_body(x_ref, w_ref, deg_ref, out_ref):
    dinv = lax.rsqrt(deg_ref[0] + deg_ref[1] + 1.0)
    h = jnp.dot(x_ref[...], w_ref[...], preferred_element_type=jnp.float32)
    g = h * dinv[:, None]
    out_ref[0] = g[:, : DH // 2]
    out_ref[1] = g[:, DH // 2:]


def _tc_a(x, w1, degp):
    return pl.pallas_call(
        _tc_a_body,
        grid=(_GRID,),
        in_specs=[
            pl.BlockSpec((_ROWS_BLK, DIN), lambda i: (i, 0)),
            pl.BlockSpec((DIN, DH), lambda i: (0, 0)),
            pl.BlockSpec((NC, _ROWS_BLK), lambda i: (0, i)),
        ],
        out_specs=pl.BlockSpec((NC, _ROWS_BLK, DH // 2), lambda i: (0, i, 0)),
        out_shape=jax.ShapeDtypeStruct((NC, N, DH // 2), jnp.float32),
    )(x, w1, degp)


def _tc_b_body(acc_ref, g_ref, deg_ref, b1_ref, w2_ref, out_ref):
    dinv = lax.rsqrt(deg_ref[0] + deg_ref[1] + 1.0)
    f = jnp.concatenate([acc_ref[0] + g_ref[0], acc_ref[1] + g_ref[1]], axis=1)
    z = jnp.maximum(f * dinv[:, None] + b1_ref[...], 0.0)
    h2 = jnp.dot(z, w2_ref[...], preferred_element_type=jnp.float32)
    out_ref[...] = h2 * dinv[:, None]


def _tc_b(acc1p, g1p, degp, b1, w2):
    return pl.pallas_call(
        _tc_b_body,
        grid=(_GRID,),
        in_specs=[
            pl.BlockSpec((NC, _ROWS_BLK, DH // 2), lambda i: (0, i, 0)),
            pl.BlockSpec((NC, _ROWS_BLK, DH // 2), lambda i: (0, i, 0)),
            pl.BlockSpec((NC, _ROWS_BLK), lambda i: (0, i)),
            pl.BlockSpec((DH,), lambda i: (0,)),
            pl.BlockSpec((DH, DOUT), lambda i: (0, 0)),
        ],
        out_specs=pl.BlockSpec((_ROWS_BLK, DOUT), lambda i: (i, 0)),
        out_shape=jax.ShapeDtypeStruct((N, DOUT), jnp.float32),
    )(acc1p, g1p, degp, b1, w2)


def _tc_c_body(acc_ref, g_ref, deg_ref, b2_ref, out_ref):
    dinv = lax.rsqrt(deg_ref[0] + deg_ref[1] + 1.0)
    f = acc_ref[0] + acc_ref[1] + g_ref[...]
    out_ref[...] = f * dinv[:, None] + b2_ref[...]


def _tc_c(acc2p, g2, degp, b2):
    return pl.pallas_call(
        _tc_c_body,
        grid=(_GRID,),
        in_specs=[
            pl.BlockSpec((NC, _ROWS_BLK, DOUT), lambda i: (0, i, 0)),
            pl.BlockSpec((_ROWS_BLK, DOUT), lambda i: (i, 0)),
            pl.BlockSpec((NC, _ROWS_BLK), lambda i: (0, i)),
            pl.BlockSpec((DOUT,), lambda i: (0,)),
        ],
        out_specs=pl.BlockSpec((_ROWS_BLK, DOUT), lambda i: (i, 0)),
        out_shape=jax.ShapeDtypeStruct((N, DOUT), jnp.float32),
    )(acc2p, g2, degp, b2)


@jax.jit
def kernel(x, edge_index, W1, b1, W2, b2):
    ei = edge_index.astype(jnp.int32)
    src = ei[0]
    dst = ei[1]
    npad = E_PAD - E
    pad_i = jnp.arange(npad, dtype=jnp.int32)
    pad_src = (pad_i * 7919) % N
    pad_dst = N + pad_i % PAD_SPREAD
    srcp = jnp.concatenate([src, pad_src]).reshape(N_CHUNKS, CHUNK)
    dstp = jnp.concatenate([dst, pad_dst]).reshape(N_CHUNKS, CHUNK)

    ones128 = jnp.ones((CHUNK,), jnp.float32)
    z640 = jnp.zeros((ROWS_PER_TILE,), jnp.float32)
    z640a = jnp.zeros((ROWS_PER_TILE, DH // 2), jnp.float32)
    z640b = jnp.zeros((ROWS_PER_TILE, DOUT), jnp.float32)

    srcp_hi = srcp + N

    degp = _deg_kernel(dstp, ones128, z640).reshape(NC, ACC_ROWS)
    g1p = _tc_a(x, W1, degp)
    acc1p = _scatter_l1(g1p.reshape(NC * N, DH // 2), srcp, srcp_hi, dstp, z640a)
    g2 = _tc_b(acc1p, g1p, degp, b1, W2)
    acc2p = _scatter_l2(g2, srcp, srcp, dstp, z640b)
    return _tc_c(acc2p, g2, degp, b2)


# speedup vs baseline: 24.0693x; 1.0025x over previous
import functools

import jax
import jax.numpy as jnp
from jax import lax
from jax.experimental import pallas as pl
from jax.experimental.pallas import tpu as pltpu
from jax.experimental.pallas import tpu_sc as plsc

N = 10000
E = 320000
DIN = 128
DH = 256
DOUT = 128

NC = 2
NS = 16
CHUNK = 128

CHUNKS_PER_TILE = 160
E_PAD = NS * CHUNKS_PER_TILE * CHUNK
N_CHUNKS = E_PAD // CHUNK
DEG_CHUNKS = N_CHUNKS // (NC * NS)

GROUP = 40

ACC_ROWS = 10240
ROWS_PER_TILE = ACC_ROWS // NS
PAD_SPREAD = ACC_ROWS - N

_mesh = functools.partial(
    plsc.VectorSubcoreMesh,
    core_axis_name="c", subcore_axis_name="s", num_cores=NC, num_subcores=NS,
)


@functools.partial(
    pl.kernel,
    out_type=jax.ShapeDtypeStruct((NC * ACC_ROWS,), jnp.float32),
    mesh=_mesh(),
    scratch_types=[
        pltpu.VMEM_SHARED((ACC_ROWS,), jnp.float32),
        pltpu.VMEM((DEG_CHUNKS, CHUNK), jnp.int32),
        pltpu.VMEM((CHUNK,), jnp.float32),
    ],
)
def _deg_kernel(dst_hbm, ones_hbm, zeros_hbm, deg_out, deg_sh, idx_v, ones_v):
    c = lax.axis_index("c")
    s = lax.axis_index("s")
    w = c * NS + s
    pltpu.sync_copy(zeros_hbm, deg_sh.at[pl.ds(s * ROWS_PER_TILE, ROWS_PER_TILE)])
    pltpu.sync_copy(ones_hbm, ones_v)
    pltpu.sync_copy(dst_hbm.at[pl.ds(w * DEG_CHUNKS, DEG_CHUNKS)], idx_v)
    plsc.subcore_barrier()

    @pl.loop(0, DEG_CHUNKS)
    def _(j):
        pltpu.sync_copy(ones_v, deg_sh.at[idx_v.at[j]], add=True)

    plsc.subcore_barrier()
    sl = pl.ds(s * ROWS_PER_TILE, ROWS_PER_TILE)
    pltpu.sync_copy(deg_sh.at[sl],
                    deg_out.at[pl.ds(c * ACC_ROWS + s * ROWS_PER_TILE, ROWS_PER_TILE)])


def _make_scatter_kernel(F, per_tile_chunks, core_split):
    groups = per_tile_chunks // GROUP

    @functools.partial(
        pl.kernel,
        out_type=jax.ShapeDtypeStruct((NC, ACC_ROWS, F), jnp.float32),
        mesh=_mesh(),
        scratch_types=[
            pltpu.VMEM_SHARED((ACC_ROWS, F), jnp.float32),
            pltpu.VMEM((GROUP, CHUNK), jnp.int32),
            pltpu.VMEM((GROUP, CHUNK), jnp.int32),
            pltpu.VMEM((2, CHUNK, F), jnp.float32),
            pltpu.SemaphoreType.DMA((2,)),
            pltpu.SemaphoreType.DMA((2,)),
            pltpu.SemaphoreType.DMA,
        ],
    )
    def scatter_kernel(tab_hbm, sa_hbm, sb_hbm, dst_hbm, zeros_hbm, out_hbm,
                       acc_sh, src_v, dst_v, rows_v, gsem, ssem, zsem):
        c = lax.axis_index("c")
        s = lax.axis_index("s")
        if core_split:
            tile0 = (c * NS + s) * per_tile_chunks
        else:
            tile0 = s * per_tile_chunks
        zd = pltpu.async_copy(
            zeros_hbm, acc_sh.at[pl.ds(s * ROWS_PER_TILE, ROWS_PER_TILE)], zsem)

        @pl.when(c == 0)
        def _():
            pltpu.sync_copy(sa_hbm.at[pl.ds(tile0, GROUP)], src_v)

        @pl.when(c == 1)
        def _():
            pltpu.sync_copy(sb_hbm.at[pl.ds(tile0, GROUP)], src_v)

        pltpu.sync_copy(dst_hbm.at[pl.ds(tile0, GROUP)], dst_v)
        pltpu.async_copy(tab_hbm.at[src_v.at[0]], rows_v.at[0], gsem.at[0])
        zd.wait()
        plsc.subcore_barrier()

        @pl.loop(0, groups)
        def _(g):
            cb = tile0 + g * GROUP

            @pl.when(g > 0)
            def _():
                @pl.when(c == 0)
                def _():
                    pltpu.sync_copy(sa_hbm.at[pl.ds(cb, GROUP)], src_v)

                @pl.when(c == 1)
                def _():
                    pltpu.sync_copy(sb_hbm.at[pl.ds(cb, GROUP)], src_v)

                pltpu.sync_copy(dst_hbm.at[pl.ds(cb, GROUP)], dst_v)
                pltpu.async_copy(tab_hbm.at[src_v.at[0]], rows_v.at[0],
                                 gsem.at[0])

            gd = [pltpu.make_async_copy(tab_hbm.at[src_v.at[0]], rows_v.at[0],
                                        gsem.at[0]), None]
            sd = [None, None]
            for jj in range(GROUP):
                b = jj & 1
                nb = 1 - b
                gd[b].wait()
                if jj + 1 < GROUP:
                    if jj >= 1:
                        sd[nb].wait()
                    gd[nb] = pltpu.async_copy(tab_hbm.at[src_v.at[jj + 1]],
                                              rows_v.at[nb], gsem.at[nb])
                sd[b] = pltpu.async_copy(rows_v.at[b], acc_sh.at[dst_v.at[jj]],
                                         ssem.at[b], add=True)
            sd[0].wait()
            sd[1].wait()

        plsc.subcore_barrier()
        sl = pl.ds(s * ROWS_PER_TILE, ROWS_PER_TILE)
        pltpu.sync_copy(acc_sh.at[sl], out_hbm.at[c, sl])

    return scatter_kernel


_scatter_l1 = _make_scatter_kernel(DH // 2, CHUNKS_PER_TILE, core_split=False)
_scatter_l2 = _make_scatter_kernel(DOUT, N_CHUNKS // (NC * NS), core_split=True)


_ROWS_BLK = 2560
_GRID = (N + _ROWS_BLK - 1) // _ROWS_BLK


def _tc_mm_body(x_ref, w_ref, out_ref):
    out_ref[...] = jnp.dot(x_ref[...], w_ref[...],
                           preferred_element_type=jnp.float32)


def _tc_mm(x, w1):
    return pl.pallas_call(
        _tc_mm_body,
        grid=(_GRID,),
        in_specs=[
            pl.BlockSpec((_ROWS_BLK, DIN), lambda i: (i, 0)),
            pl.BlockSpec((DIN, DH), lambda i: (0, 0)),
        ],
        out_specs=pl.BlockSpec((_ROWS_BLK, DH), lambda i: (i, 0)),
        out_shape=jax.ShapeDtypeStruct((N, DH), jnp.float32),
    )(x, w1)


def _tc_a_body(h_ref, deg_ref, out_ref):
    dinv = lax.rsqrt(deg_ref[0] + deg_ref[1] + 1.0)
    g = h_ref[...] * dinv[:, None]
    out_ref[0] = g[:, : DH // 2]
    out_ref[1] = g[:, DH // 2:]


def _tc_a(h, degp):
    return pl.pallas_call(
        _tc_a_body,
        grid=(_GRID,),
        in_specs=[
            pl.BlockSpec((_ROWS_BLK, DH), lambda i: (i, 0)),
            pl.BlockSpec((NC, _ROWS_BLK), lambda i: (0, i)),
        ],
        out_specs=pl.BlockSpec((NC, _ROWS_BLK, DH // 2), lambda i: (0, i, 0)),
        out_shape=jax.ShapeDtypeStruct((NC, N, DH // 2), jnp.float32),
    )(h, degp)


def _tc_b_body(acc_ref, g_ref, deg_ref, b1_ref, w2_ref, out_ref):
    dinv = lax.rsqrt(deg_ref[0] + deg_ref[1] + 1.0)
    f = jnp.concatenate([acc_ref[0] + g_ref[0], acc_ref[1] + g_ref[1]], axis=1)
    z = jnp.maximum(f * dinv[:, None] + b1_ref[...], 0.0)
    h2 = jnp.dot(z, w2_ref[...], preferred_element_type=jnp.float32)
    out_ref[...] = h2 * dinv[:, None]


def _tc_b(acc1p, g1p, degp, b1, w2):
    return pl.pallas_call(
        _tc_b_body,
        grid=(_GRID,),
        in_specs=[
            pl.BlockSpec((NC, _ROWS_BLK, DH // 2), lambda i: (0, i, 0)),
            pl.BlockSpec((NC, _ROWS_BLK, DH // 2), lambda i: (0, i, 0)),
            pl.BlockSpec((NC, _ROWS_BLK), lambda i: (0, i)),
            pl.BlockSpec((DH,), lambda i: (0,)),
            pl.BlockSpec((DH, DOUT), lambda i: (0, 0)),
        ],
        out_specs=pl.BlockSpec((_ROWS_BLK, DOUT), lambda i: (i, 0)),
        out_shape=jax.ShapeDtypeStruct((N, DOUT), jnp.float32),
    )(acc1p, g1p, degp, b1, w2)


def _tc_c_body(acc_ref, g_ref, deg_ref, b2_ref, out_ref):
    dinv = lax.rsqrt(deg_ref[0] + deg_ref[1] + 1.0)
    f = acc_ref[0] + acc_ref[1] + g_ref[...]
    out_ref[...] = f * dinv[:, None] + b2_ref[...]


def _tc_c(acc2p, g2, degp, b2):
    return pl.pallas_call(
        _tc_c_body,
        grid=(_GRID,),
        in_specs=[
            pl.BlockSpec((NC, _ROWS_BLK, DOUT), lambda i: (0, i, 0)),
            pl.BlockSpec((_ROWS_BLK, DOUT), lambda i: (i, 0)),
            pl.BlockSpec((NC, _ROWS_BLK), lambda i: (0, i)),
            pl.BlockSpec((DOUT,), lambda i: (0,)),
        ],
        out_specs=pl.BlockSpec((_ROWS_BLK, DOUT), lambda i: (i, 0)),
        out_shape=jax.ShapeDtypeStruct((N, DOUT), jnp.float32),
    )(acc2p, g2, degp, b2)


@jax.jit
def kernel(x, edge_index, W1, b1, W2, b2):
    ei = edge_index.astype(jnp.int32)
    src = ei[0]
    dst = ei[1]
    npad = E_PAD - E
    pad_i = jnp.arange(npad, dtype=jnp.int32)
    pad_src = (pad_i * 7919) % N
    pad_dst = N + pad_i % PAD_SPREAD
    srcp = jnp.concatenate([src, pad_src]).reshape(N_CHUNKS, CHUNK)
    dstp = jnp.concatenate([dst, pad_dst]).reshape(N_CHUNKS, CHUNK)

    ones128 = jnp.ones((CHUNK,), jnp.float32)
    z640 = jnp.zeros((ROWS_PER_TILE,), jnp.float32)
    z640a = jnp.zeros((ROWS_PER_TILE, DH // 2), jnp.float32)
    z640b = jnp.zeros((ROWS_PER_TILE, DOUT), jnp.float32)

    srcp_hi = srcp + N

    h1 = _tc_mm(x, W1)
    degp = _deg_kernel(dstp, ones128, z640).reshape(NC, ACC_ROWS)
    g1p = _tc_a(h1, degp)
    acc1p = _scatter_l1(g1p.reshape(NC * N, DH // 2), srcp, srcp_hi, dstp, z640a)
    g2 = _tc_b(acc1p, g1p, degp, b1, W2)
    acc2p = _scatter_l2(g2, srcp, srcp, dstp, z640b)
    return _tc_c(acc2p, g2, degp, b2)


# speedup vs baseline: 24.1185x; 1.0020x over previous
import functools

import jax
import jax.numpy as jnp
from jax import lax
from jax.experimental import pallas as pl
from jax.experimental.pallas import tpu as pltpu
from jax.experimental.pallas import tpu_sc as plsc

N = 10000
E = 320000
DIN = 128
DH = 256
DOUT = 128

NC = 2
NS = 16
CHUNK = 128

CHUNKS_PER_TILE = 160
E_PAD = NS * CHUNKS_PER_TILE * CHUNK
N_CHUNKS = E_PAD // CHUNK
DEG_CHUNKS = N_CHUNKS // (NC * NS)

GROUP = 40

ACC_ROWS = 10240
ROWS_PER_TILE = ACC_ROWS // NS
PAD_SPREAD = ACC_ROWS - N

_mesh = functools.partial(
    plsc.VectorSubcoreMesh,
    core_axis_name="c", subcore_axis_name="s", num_cores=NC, num_subcores=NS,
)


@functools.partial(
    pl.kernel,
    out_type=jax.ShapeDtypeStruct((NC * ACC_ROWS,), jnp.float32),
    mesh=_mesh(),
    scratch_types=[
        pltpu.VMEM_SHARED((ACC_ROWS,), jnp.float32),
        pltpu.VMEM((DEG_CHUNKS, CHUNK), jnp.int32),
        pltpu.VMEM((CHUNK,), jnp.float32),
    ],
)
def _deg_kernel(dst_hbm, ones_hbm, zeros_hbm, deg_out, deg_sh, idx_v, ones_v):
    c = lax.axis_index("c")
    s = lax.axis_index("s")
    w = c * NS + s
    pltpu.sync_copy(zeros_hbm, deg_sh.at[pl.ds(s * ROWS_PER_TILE, ROWS_PER_TILE)])
    pltpu.sync_copy(ones_hbm, ones_v)
    pltpu.sync_copy(dst_hbm.at[pl.ds(w * DEG_CHUNKS, DEG_CHUNKS)], idx_v)
    plsc.subcore_barrier()

    @pl.loop(0, DEG_CHUNKS)
    def _(j):
        pltpu.sync_copy(ones_v, deg_sh.at[idx_v.at[j]], add=True)

    plsc.subcore_barrier()
    sl = pl.ds(s * ROWS_PER_TILE, ROWS_PER_TILE)
    pltpu.sync_copy(deg_sh.at[sl],
                    deg_out.at[pl.ds(c * ACC_ROWS + s * ROWS_PER_TILE, ROWS_PER_TILE)])


def _make_scatter_kernel(F, per_tile_chunks, core_split):
    groups = per_tile_chunks // GROUP

    @functools.partial(
        pl.kernel,
        out_type=jax.ShapeDtypeStruct((NC, ACC_ROWS, F), jnp.float32),
        mesh=_mesh(),
        scratch_types=[
            pltpu.VMEM_SHARED((ACC_ROWS, F), jnp.float32),
            pltpu.VMEM((GROUP, CHUNK), jnp.int32),
            pltpu.VMEM((GROUP, CHUNK), jnp.int32),
            pltpu.VMEM((2, CHUNK, F), jnp.float32),
            pltpu.SemaphoreType.DMA((2,)),
            pltpu.SemaphoreType.DMA((2,)),
            pltpu.SemaphoreType.DMA,
        ],
    )
    def scatter_kernel(tab_hbm, sa_hbm, sb_hbm, dst_hbm, zeros_hbm, out_hbm,
                       acc_sh, src_v, dst_v, rows_v, gsem, ssem, zsem):
        c = lax.axis_index("c")
        s = lax.axis_index("s")
        if core_split:
            tile0 = (c * NS + s) * per_tile_chunks
        else:
            tile0 = s * per_tile_chunks
        zd = pltpu.async_copy(
            zeros_hbm, acc_sh.at[pl.ds(s * ROWS_PER_TILE, ROWS_PER_TILE)], zsem)

        @pl.when(c == 0)
        def _():
            pltpu.sync_copy(sa_hbm.at[pl.ds(tile0, GROUP)], src_v)

        @pl.when(c == 1)
        def _():
            pltpu.sync_copy(sb_hbm.at[pl.ds(tile0, GROUP)], src_v)

        pltpu.sync_copy(dst_hbm.at[pl.ds(tile0, GROUP)], dst_v)
        pltpu.async_copy(tab_hbm.at[src_v.at[0]], rows_v.at[0], gsem.at[0])
        zd.wait()
        plsc.subcore_barrier()

        @pl.loop(0, groups)
        def _(g):
            cb = tile0 + g * GROUP

            @pl.when(g > 0)
            def _():
                @pl.when(c == 0)
                def _():
                    pltpu.sync_copy(sa_hbm.at[pl.ds(cb, GROUP)], src_v)

                @pl.when(c == 1)
                def _():
                    pltpu.sync_copy(sb_hbm.at[pl.ds(cb, GROUP)], src_v)

                pltpu.sync_copy(dst_hbm.at[pl.ds(cb, GROUP)], dst_v)
                pltpu.async_copy(tab_hbm.at[src_v.at[0]], rows_v.at[0],
                                 gsem.at[0])

            gd = [pltpu.make_async_copy(tab_hbm.at[src_v.at[0]], rows_v.at[0],
                                        gsem.at[0]), None]
            sd = [None, None]
            for jj in range(GROUP):
                b = jj & 1
                nb = 1 - b
                gd[b].wait()
                if jj + 1 < GROUP:
                    if jj >= 1:
                        sd[nb].wait()
                    gd[nb] = pltpu.async_copy(tab_hbm.at[src_v.at[jj + 1]],
                                              rows_v.at[nb], gsem.at[nb])
                sd[b] = pltpu.async_copy(rows_v.at[b], acc_sh.at[dst_v.at[jj]],
                                         ssem.at[b], priority=1, add=True)
            sd[0].wait()
            sd[1].wait()

        plsc.subcore_barrier()
        sl = pl.ds(s * ROWS_PER_TILE, ROWS_PER_TILE)
        pltpu.sync_copy(acc_sh.at[sl], out_hbm.at[c, sl])

    return scatter_kernel


_scatter_l1 = _make_scatter_kernel(DH // 2, CHUNKS_PER_TILE, core_split=False)
_scatter_l2 = _make_scatter_kernel(DOUT, N_CHUNKS // (NC * NS), core_split=True)


_ROWS_BLK = 2560
_GRID = (N + _ROWS_BLK - 1) // _ROWS_BLK


def _tc_mm_body(x_ref, w_ref, out_ref):
    out_ref[...] = jnp.dot(x_ref[...], w_ref[...],
                           preferred_element_type=jnp.float32)


def _tc_mm(x, w1):
    return pl.pallas_call(
        _tc_mm_body,
        grid=(_GRID,),
        in_specs=[
            pl.BlockSpec((_ROWS_BLK, DIN), lambda i: (i, 0)),
            pl.BlockSpec((DIN, DH), lambda i: (0, 0)),
        ],
        out_specs=pl.BlockSpec((_ROWS_BLK, DH), lambda i: (i, 0)),
        out_shape=jax.ShapeDtypeStruct((N, DH), jnp.float32),
    )(x, w1)


def _tc_a_body(h_ref, deg_ref, out_ref):
    dinv = lax.rsqrt(deg_ref[0] + deg_ref[1] + 1.0)
    g = h_ref[...] * dinv[:, None]
    out_ref[0] = g[:, : DH // 2]
    out_ref[1] = g[:, DH // 2:]


def _tc_a(h, degp):
    return pl.pallas_call(
        _tc_a_body,
        grid=(_GRID,),
        in_specs=[
            pl.BlockSpec((_ROWS_BLK, DH), lambda i: (i, 0)),
            pl.BlockSpec((NC, _ROWS_BLK), lambda i: (0, i)),
        ],
        out_specs=pl.BlockSpec((NC, _ROWS_BLK, DH // 2), lambda i: (0, i, 0)),
        out_shape=jax.ShapeDtypeStruct((NC, N, DH // 2), jnp.float32),
    )(h, degp)


def _tc_b_body(acc_ref, g_ref, deg_ref, b1_ref, w2_ref, out_ref):
    dinv = lax.rsqrt(deg_ref[0] + deg_ref[1] + 1.0)
    f = jnp.concatenate([acc_ref[0] + g_ref[0], acc_ref[1] + g_ref[1]], axis=1)
    z = jnp.maximum(f * dinv[:, None] + b1_ref[...], 0.0)
    h2 = jnp.dot(z, w2_ref[...], preferred_element_type=jnp.float32)
    out_ref[...] = h2 * dinv[:, None]


def _tc_b(acc1p, g1p, degp, b1, w2):
    return pl.pallas_call(
        _tc_b_body,
        grid=(_GRID,),
        in_specs=[
            pl.BlockSpec((NC, _ROWS_BLK, DH // 2), lambda i: (0, i, 0)),
            pl.BlockSpec((NC, _ROWS_BLK, DH // 2), lambda i: (0, i, 0)),
            pl.BlockSpec((NC, _ROWS_BLK), lambda i: (0, i)),
            pl.BlockSpec((DH,), lambda i: (0,)),
            pl.BlockSpec((DH, DOUT), lambda i: (0, 0)),
        ],
        out_specs=pl.BlockSpec((_ROWS_BLK, DOUT), lambda i: (i, 0)),
        out_shape=jax.ShapeDtypeStruct((N, DOUT), jnp.float32),
    )(acc1p, g1p, degp, b1, w2)


def _tc_c_body(acc_ref, g_ref, deg_ref, b2_ref, out_ref):
    dinv = lax.rsqrt(deg_ref[0] + deg_ref[1] + 1.0)
    f = acc_ref[0] + acc_ref[1] + g_ref[...]
    out_ref[...] = f * dinv[:, None] + b2_ref[...]


def _tc_c(acc2p, g2, degp, b2):
    return pl.pallas_call(
        _tc_c_body,
        grid=(_GRID,),
        in_specs=[
            pl.BlockSpec((NC, _ROWS_BLK, DOUT), lambda i: (0, i, 0)),
            pl.BlockSpec((_ROWS_BLK, DOUT), lambda i: (i, 0)),
            pl.BlockSpec((NC, _ROWS_BLK), lambda i: (0, i)),
            pl.BlockSpec((DOUT,), lambda i: (0,)),
        ],
        out_specs=pl.BlockSpec((_ROWS_BLK, DOUT), lambda i: (i, 0)),
        out_shape=jax.ShapeDtypeStruct((N, DOUT), jnp.float32),
    )(acc2p, g2, degp, b2)


@jax.jit
def kernel(x, edge_index, W1, b1, W2, b2):
    ei = edge_index.astype(jnp.int32)
    src = ei[0]
    dst = ei[1]
    npad = E_PAD - E
    pad_i = jnp.arange(npad, dtype=jnp.int32)
    pad_src = (pad_i * 7919) % N
    pad_dst = N + pad_i % PAD_SPREAD
    srcp = jnp.concatenate([src, pad_src]).reshape(N_CHUNKS, CHUNK)
    dstp = jnp.concatenate([dst, pad_dst]).reshape(N_CHUNKS, CHUNK)

    ones128 = jnp.ones((CHUNK,), jnp.float32)
    z640 = jnp.zeros((ROWS_PER_TILE,), jnp.float32)
    z640a = jnp.zeros((ROWS_PER_TILE, DH // 2), jnp.float32)
    z640b = jnp.zeros((ROWS_PER_TILE, DOUT), jnp.float32)

    srcp_hi = srcp + N

    h1 = _tc_mm(x, W1)
    degp = _deg_kernel(dstp, ones128, z640).reshape(NC, ACC_ROWS)
    g1p = _tc_a(h1, degp)
    acc1p = _scatter_l1(g1p.reshape(NC * N, DH // 2), srcp, srcp_hi, dstp, z640a)
    g2 = _tc_b(acc1p, g1p, degp, b1, W2)
    acc2p = _scatter_l2(g2, srcp, srcp, dstp, z640b)
    return _tc_c(acc2p, g2, degp, b2)


# speedup vs baseline: 24.1406x; 1.0009x over previous
import functools

import jax
import jax.numpy as jnp
from jax import lax
from jax.experimental import pallas as pl
from jax.experimental.pallas import tpu as pltpu
from jax.experimental.pallas import tpu_sc as plsc

N = 10000
E = 320000
DIN = 128
DH = 256
DOUT = 128

NC = 2
NS = 16
CHUNK = 128

CHUNKS_PER_TILE = 160
E_PAD = NS * CHUNKS_PER_TILE * CHUNK
N_CHUNKS = E_PAD // CHUNK
DEG_CHUNKS = N_CHUNKS // (NC * NS)

GROUP = 40

ACC_ROWS = 10240
ROWS_PER_TILE = ACC_ROWS // NS
PAD_SPREAD = ACC_ROWS - N

_mesh = functools.partial(
    plsc.VectorSubcoreMesh,
    core_axis_name="c", subcore_axis_name="s", num_cores=NC, num_subcores=NS,
)


@functools.partial(
    pl.kernel,
    out_type=jax.ShapeDtypeStruct((NC * ACC_ROWS,), jnp.float32),
    mesh=_mesh(),
    scratch_types=[
        pltpu.VMEM_SHARED((ACC_ROWS,), jnp.float32),
        pltpu.VMEM((DEG_CHUNKS, CHUNK), jnp.int32),
        pltpu.VMEM((CHUNK,), jnp.float32),
    ],
)
def _deg_kernel(dst_hbm, ones_hbm, zeros_hbm, deg_out, deg_sh, idx_v, ones_v):
    c = lax.axis_index("c")
    s = lax.axis_index("s")
    w = c * NS + s
    pltpu.sync_copy(zeros_hbm, deg_sh.at[pl.ds(s * ROWS_PER_TILE, ROWS_PER_TILE)])
    pltpu.sync_copy(ones_hbm, ones_v)
    pltpu.sync_copy(dst_hbm.at[pl.ds(w * DEG_CHUNKS, DEG_CHUNKS)], idx_v)
    plsc.subcore_barrier()

    @pl.loop(0, DEG_CHUNKS)
    def _(j):
        pltpu.sync_copy(ones_v, deg_sh.at[idx_v.at[j]], add=True)

    plsc.subcore_barrier()
    sl = pl.ds(s * ROWS_PER_TILE, ROWS_PER_TILE)
    pltpu.sync_copy(deg_sh.at[sl],
                    deg_out.at[pl.ds(c * ACC_ROWS + s * ROWS_PER_TILE, ROWS_PER_TILE)])


def _make_scatter_kernel(F, per_tile_chunks, core_split):
    groups = per_tile_chunks // GROUP

    @functools.partial(
        pl.kernel,
        out_type=jax.ShapeDtypeStruct((NC, ACC_ROWS, F), jnp.float32),
        mesh=_mesh(),
        scratch_types=[
            pltpu.VMEM_SHARED((ACC_ROWS, F), jnp.float32),
            pltpu.VMEM((GROUP, CHUNK), jnp.int32),
            pltpu.VMEM((GROUP, CHUNK), jnp.int32),
            pltpu.VMEM((2, CHUNK, F), jnp.float32),
            pltpu.SemaphoreType.DMA((2,)),
            pltpu.SemaphoreType.DMA((2,)),
            pltpu.SemaphoreType.DMA,
        ],
    )
    def scatter_kernel(tab_hbm, sa_hbm, sb_hbm, dst_hbm, zeros_hbm, out_hbm,
                       acc_sh, src_v, dst_v, rows_v, gsem, ssem, zsem):
        c = lax.axis_index("c")
        s = lax.axis_index("s")
        if core_split:
            tile0 = (c * NS + s) * per_tile_chunks
        else:
            tile0 = s * per_tile_chunks
        zd = pltpu.async_copy(
            zeros_hbm, acc_sh.at[pl.ds(s * ROWS_PER_TILE, ROWS_PER_TILE)], zsem)

        @pl.when(c == 0)
        def _():
            pltpu.sync_copy(sa_hbm.at[pl.ds(tile0, GROUP)], src_v)

        @pl.when(c == 1)
        def _():
            pltpu.sync_copy(sb_hbm.at[pl.ds(tile0, GROUP)], src_v)

        pltpu.sync_copy(dst_hbm.at[pl.ds(tile0, GROUP)], dst_v)
        pltpu.async_copy(tab_hbm.at[src_v.at[0]], rows_v.at[0], gsem.at[0])
        zd.wait()
        plsc.subcore_barrier()

        @pl.loop(0, groups)
        def _(g):
            cb = tile0 + g * GROUP

            @pl.when(g > 0)
            def _():
                @pl.when(c == 0)
                def _():
                    pltpu.sync_copy(sa_hbm.at[pl.ds(cb, GROUP)], src_v)

                @pl.when(c == 1)
                def _():
                    pltpu.sync_copy(sb_hbm.at[pl.ds(cb, GROUP)], src_v)

                pltpu.sync_copy(dst_hbm.at[pl.ds(cb, GROUP)], dst_v)
                pltpu.async_copy(tab_hbm.at[src_v.at[0]], rows_v.at[0],
                                 gsem.at[0])

            gd = [pltpu.make_async_copy(tab_hbm.at[src_v.at[0]], rows_v.at[0],
                                        gsem.at[0]), None]
            sd = [None, None]
            for jj in range(GROUP):
                b = jj & 1
                nb = 1 - b
                gd[b].wait()
                if jj + 1 < GROUP:
                    if jj >= 1:
                        sd[nb].wait()
                    gd[nb] = pltpu.async_copy(tab_hbm.at[src_v.at[jj + 1]],
                                              rows_v.at[nb], gsem.at[nb])
                sd[b] = pltpu.async_copy(rows_v.at[b], acc_sh.at[dst_v.at[jj]],
                                         ssem.at[b], add=True)
            sd[0].wait()
            sd[1].wait()

        plsc.subcore_barrier()
        sl = pl.ds(s * ROWS_PER_TILE, ROWS_PER_TILE)
        pltpu.sync_copy(acc_sh.at[sl], out_hbm.at[c, sl])

    return scatter_kernel


_scatter_l1 = _make_scatter_kernel(DH // 2, CHUNKS_PER_TILE, core_split=False)
_scatter_l2 = _make_scatter_kernel(DOUT, N_CHUNKS // (NC * NS), core_split=True)


_ROWS_BLK = 2560
_GRID = (N + _ROWS_BLK - 1) // _ROWS_BLK


def _tc_mm_body(x_ref, w_ref, out_ref):
    out_ref[...] = jnp.dot(x_ref[...], w_ref[...],
                           preferred_element_type=jnp.float32)


def _tc_mm(x, w1):
    return pl.pallas_call(
        _tc_mm_body,
        grid=(_GRID,),
        in_specs=[
            pl.BlockSpec((_ROWS_BLK, DIN), lambda i: (i, 0)),
            pl.BlockSpec((DIN, DH), lambda i: (0, 0)),
        ],
        out_specs=pl.BlockSpec((_ROWS_BLK, DH), lambda i: (i, 0)),
        out_shape=jax.ShapeDtypeStruct((N, DH), jnp.float32),
    )(x, w1)


def _tc_a_body(h_ref, deg_ref, out_ref):
    dinv = lax.rsqrt(deg_ref[0] + deg_ref[1] + 1.0)
    g = h_ref[...] * dinv[:, None]
    out_ref[0] = g[:, : DH // 2]
    out_ref[1] = g[:, DH // 2:]


def _tc_a(h, degp):
    return pl.pallas_call(
        _tc_a_body,
        grid=(_GRID,),
        in_specs=[
            pl.BlockSpec((_ROWS_BLK, DH), lambda i: (i, 0)),
            pl.BlockSpec((NC, _ROWS_BLK), lambda i: (0, i)),
        ],
        out_specs=pl.BlockSpec((NC, _ROWS_BLK, DH // 2), lambda i: (0, i, 0)),
        out_shape=jax.ShapeDtypeStruct((NC, N, DH // 2), jnp.float32),
    )(h, degp)


def _tc_b_body(acc_ref, g_ref, deg_ref, b1_ref, w2_ref, out_ref):
    dinv = lax.rsqrt(deg_ref[0] + deg_ref[1] + 1.0)
    f = jnp.concatenate([acc_ref[0] + g_ref[0], acc_ref[1] + g_ref[1]], axis=1)
    z = jnp.maximum(f * dinv[:, None] + b1_ref[...], 0.0)
    h2 = jnp.dot(z, w2_ref[...], preferred_element_type=jnp.float32)
    out_ref[...] = h2 * dinv[:, None]


def _tc_b(acc1p, g1p, degp, b1, w2):
    return pl.pallas_call(
        _tc_b_body,
        grid=(_GRID,),
        in_specs=[
            pl.BlockSpec((NC, _ROWS_BLK, DH // 2), lambda i: (0, i, 0)),
            pl.BlockSpec((NC, _ROWS_BLK, DH // 2), lambda i: (0, i, 0)),
            pl.BlockSpec((NC, _ROWS_BLK), lambda i: (0, i)),
            pl.BlockSpec((DH,), lambda i: (0,)),
            pl.BlockSpec((DH, DOUT), lambda i: (0, 0)),
        ],
        out_specs=pl.BlockSpec((_ROWS_BLK, DOUT), lambda i: (i, 0)),
        out_shape=jax.ShapeDtypeStruct((N, DOUT), jnp.float32),
    )(acc1p, g1p, degp, b1, w2)


def _tc_c_body(acc_ref, g_ref, deg_ref, b2_ref, out_ref):
    dinv = lax.rsqrt(deg_ref[0] + deg_ref[1] + 1.0)
    f = acc_ref[0] + acc_ref[1] + g_ref[...]
    out_ref[...] = f * dinv[:, None] + b2_ref[...]


def _tc_c(acc2p, g2, degp, b2):
    return pl.pallas_call(
        _tc_c_body,
        grid=(_GRID,),
        in_specs=[
            pl.BlockSpec((NC, _ROWS_BLK, DOUT), lambda i: (0, i, 0)),
            pl.BlockSpec((_ROWS_BLK, DOUT), lambda i: (i, 0)),
            pl.BlockSpec((NC, _ROWS_BLK), lambda i: (0, i)),
            pl.BlockSpec((DOUT,), lambda i: (0,)),
        ],
        out_specs=pl.BlockSpec((_ROWS_BLK, DOUT), lambda i: (i, 0)),
        out_shape=jax.ShapeDtypeStruct((N, DOUT), jnp.float32),
    )(acc2p, g2, degp, b2)


@jax.jit
def kernel(x, edge_index, W1, b1, W2, b2):
    ei = edge_index.astype(jnp.int32)
    src = ei[0]
    dst = ei[1]
    npad = E_PAD - E
    pad_i = jnp.arange(npad, dtype=jnp.int32)
    pad_src = (pad_i * 7919) % N
    pad_dst = N + pad_i % PAD_SPREAD
    srcp = jnp.concatenate([src, pad_src]).reshape(N_CHUNKS, CHUNK)
    dstp = jnp.concatenate([dst, pad_dst]).reshape(N_CHUNKS, CHUNK)

    ones128 = jnp.ones((CHUNK,), jnp.float32)
    z640 = jnp.zeros((ROWS_PER_TILE,), jnp.float32)
    z640a = jnp.zeros((ROWS_PER_TILE, DH // 2), jnp.float32)
    z640b = jnp.zeros((ROWS_PER_TILE, DOUT), jnp.float32)

    srcp_hi = srcp + N

    h1 = _tc_mm(x, W1)
    degp = _deg_kernel(dstp, ones128, z640).reshape(NC, ACC_ROWS)
    g1p = _tc_a(h1, degp)
    acc1p = _scatter_l1(g1p.reshape(NC * N, DH // 2), srcp, srcp_hi, dstp, z640a)
    g2 = _tc_b(acc1p, g1p, degp, b1, W2)
    acc2p = _scatter_l2(g2, srcp, srcp, dstp, z640b)
    return _tc_c(acc2p, g2, degp, b2)
